# Initial kernel scaffold; baseline (speedup 1.0000x reference)
import jax
import jax.numpy as jnp
from jax.experimental import pallas as pl


def kernel(features, edges, Wl1, bl1, Wr1, Wl_mu, bl_mu, Wr_mu, Wl_std, bl_std, Wr_std):
    raise NotImplementedError("write your pallas kernel here")


# speedup vs baseline: 5.6933x; 5.6933x over previous
import dataclasses

import jax
import jax.numpy as jnp
from jax import lax
from jax.experimental import pallas as pl
from jax.experimental.pallas import tpu as pltpu
from jax.experimental.pallas import tpu_sc as plsc

NN = 10000
NP = 10240
D = 128
E = 320000
CHUNK = 128
NCHUNKS = E // CHUNK
NC, NS = 2, 16
CHUNKS_PER_CORE = NCHUNKS // NC
ROWS_PER_TILE = NP // NS
ZR = 16
HR = NP // D


def _mesh():
    return plsc.VectorSubcoreMesh(core_axis_name="c", subcore_axis_name="s",
                                  num_cores=NC, num_subcores=NS)


def _no_layout_params():
    cp = pltpu.CompilerParams()
    if "needs_layout_passes" in pltpu.CompilerParams.__dataclass_fields__:
        cp = dataclasses.replace(cp, needs_layout_passes=False)
    return cp


def _chunk_bounds(c, s):
    nloc = jnp.where(s < CHUNKS_PER_CORE - NS * (CHUNKS_PER_CORE // NS),
                     CHUNKS_PER_CORE // NS + 1, CHUNKS_PER_CORE // NS)
    return c * CHUNKS_PER_CORE + s, nloc


def _sc_agg_body(x_hbm, src_hbm, dst_hbm, agg_hbm,
                 acc_sp, src_v, dst_v, rows_v, zb_v, sem):
    c = lax.axis_index("c")
    s = lax.axis_index("s")
    z16 = jnp.zeros((16,), jnp.float32)

    def zero_zb(i, _):
        for j in range(D // 16):
            zb_v[i, pl.ds(j * 16, 16)] = z16
        return _
    lax.fori_loop(0, ZR, zero_zb, None)

    base_row = s * ROWS_PER_TILE

    def zero_acc(i, _):
        pltpu.sync_copy(zb_v, acc_sp.at[pl.ds(base_row + i * ZR, ZR)])
        return _
    lax.fori_loop(0, ROWS_PER_TILE // ZR, zero_acc, None)

    plsc.subcore_barrier()

    base_chunk, nloc = _chunk_bounds(c, s)

    def edge_chunk(i, _):
        off = (base_chunk + i * NS) * CHUNK
        pltpu.sync_copy(src_hbm.at[pl.ds(off, CHUNK)], src_v)
        pltpu.sync_copy(dst_hbm.at[pl.ds(off, CHUNK)], dst_v)
        pltpu.async_copy(x_hbm.at[src_v], rows_v, sem).wait()
        pltpu.sync_copy(rows_v, acc_sp.at[dst_v], add=True)
        return _
    lax.fori_loop(0, nloc, edge_chunk, None)

    plsc.subcore_barrier()

    pltpu.sync_copy(acc_sp.at[pl.ds(base_row, ROWS_PER_TILE)],
                    agg_hbm.at[c, pl.ds(base_row, ROWS_PER_TILE)])


def _sc_aggregate(x, src, dst):
    k = pl.kernel(
        _sc_agg_body,
        out_type=jax.ShapeDtypeStruct((NC, NP, D), jnp.float32),
        mesh=_mesh(),
        scratch_types=[
            pltpu.VMEM_SHARED((NP, D), jnp.float32),
            pltpu.VMEM((CHUNK,), jnp.int32),
            pltpu.VMEM((CHUNK,), jnp.int32),
            pltpu.VMEM((CHUNK, D), jnp.float32),
            pltpu.VMEM((ZR, D), jnp.float32),
            pltpu.SemaphoreType.DMA,
        ],
    )
    return k(x, src, dst)


def _sc_count_body(dst_hbm, cnt_hbm, hist_sp, dst_v, hist_v, idx80_v, sem):
    c = lax.axis_index("c")
    s = lax.axis_index("s")
    z16 = jnp.zeros((16,), jnp.float32)
    ones16 = jnp.ones((16,), jnp.float32)

    def zero_hist(i, _):
        for j in range(D // 16):
            hist_v[i, pl.ds(j * 16, 16)] = z16
        return _
    lax.fori_loop(0, HR, zero_hist, None)

    def fill_idx(i, _):
        idx80_v[pl.ds(i * 16, 16)] = lax.iota(jnp.int32, 16) + i * 16
        return _
    lax.fori_loop(0, HR // 16, fill_idx, None)

    @pl.when(s < HR // 8)
    def _():
        pltpu.sync_copy(hist_v.at[pl.ds(0, 8)], hist_sp.at[pl.ds(s * 8, 8)])
    plsc.subcore_barrier()

    base_chunk, nloc = _chunk_bounds(c, s)

    def edge_chunk(i, _):
        off = (base_chunk + i * NS) * CHUNK
        pltpu.sync_copy(dst_hbm.at[pl.ds(off, CHUNK)], dst_v)
        for j in range(CHUNK // 16):
            idx = dst_v[pl.ds(j * 16, 16)]
            row = lax.shift_right_logical(idx, 7)
            lane = lax.bitwise_and(idx, 127)
            plsc.addupdate_scatter(hist_v, [row, lane], ones16)
        return _
    lax.fori_loop(0, nloc, edge_chunk, None)

    pltpu.sync_copy(hist_v, hist_sp.at[idx80_v], add=True)
    plsc.subcore_barrier()

    @pl.when(s == 0)
    def _():
        pltpu.sync_copy(hist_sp, cnt_hbm.at[c])


def _sc_count(dst):
    k = pl.kernel(
        _sc_count_body,
        out_type=jax.ShapeDtypeStruct((NC, HR, D), jnp.float32),
        mesh=_mesh(),
        scratch_types=[
            pltpu.VMEM_SHARED((HR, D), jnp.float32),
            pltpu.VMEM((CHUNK,), jnp.int32),
            pltpu.VMEM((HR, D), jnp.float32),
            pltpu.VMEM((HR,), jnp.int32),
            pltpu.SemaphoreType.DMA,
        ],
        compiler_params=_no_layout_params(),
    )
    return k(dst)


_HI = lax.Precision.HIGHEST
_BR = 1024
_GRID = NP // _BR


def _expand_cnt(pc_ref):
    pk = (pc_ref[0] + pc_ref[1]).T
    return jnp.concatenate([pk[:, g:g + 1] for g in range(_BR // D)], axis=0)


def _dense1_body(pa_ref, pc_ref, x_ref, wl_ref, bl_ref, wr_ref, h_ref):
    agg = pa_ref[0] + pa_ref[1]
    cnt = _expand_cnt(pc_ref)
    mean = agg / jnp.maximum(cnt, 1.0)
    acc = jnp.dot(mean, wl_ref[...], preferred_element_type=jnp.float32,
                  precision=_HI)
    acc = acc + jnp.dot(x_ref[...], wr_ref[...],
                        preferred_element_type=jnp.float32, precision=_HI)
    h_ref[...] = jnp.maximum(acc + bl_ref[...], 0.0)


def _dense2_body(pa_ref, pc_ref, h_ref, wl_ref, b_ref, wr_ref, o_ref):
    agg = pa_ref[0] + pa_ref[1]
    cnt = _expand_cnt(pc_ref)
    mean = agg / jnp.maximum(cnt, 1.0)
    acc = jnp.dot(mean, wl_ref[...], preferred_element_type=jnp.float32,
                  precision=_HI)
    acc = acc + jnp.dot(h_ref[...], wr_ref[...],
                        preferred_element_type=jnp.float32, precision=_HI)
    o_ref[...] = acc + b_ref[...]


def _row_specs():
    return [
        pl.BlockSpec((NC, _BR, D), lambda i: (0, i, 0)),
        pl.BlockSpec((NC, _BR // D, D), lambda i: (0, i, 0)),
        pl.BlockSpec((_BR, D), lambda i: (i, 0)),
        pl.BlockSpec((D, D), lambda i: (0, 0)),
        pl.BlockSpec((1, D), lambda i: (0, 0)),
        pl.BlockSpec((D, D), lambda i: (0, 0)),
    ]


def kernel(features, edges, Wl1, bl1, Wr1, Wl_mu, bl_mu, Wr_mu,
           Wl_std, bl_std, Wr_std):
    edges = edges.astype(jnp.int32)
    src, dst = edges[0], edges[1]
    x = jnp.pad(features, ((0, NP - NN), (0, 0)))

    pa1 = _sc_aggregate(x, src, dst)
    pc = _sc_count(dst)

    h = pl.pallas_call(
        _dense1_body,
        grid=(_GRID,),
        in_specs=_row_specs(),
        out_specs=pl.BlockSpec((_BR, D), lambda i: (i, 0)),
        out_shape=jax.ShapeDtypeStruct((NP, D), jnp.float32),
    )(pa1, pc, x, Wl1.T, bl1.reshape(1, D), Wr1.T)

    pa2 = _sc_aggregate(h, src, dst)

    wl2 = jnp.concatenate([Wl_mu.T, Wl_std.T], axis=1)
    wr2 = jnp.concatenate([Wr_mu.T, Wr_std.T], axis=1)
    b2 = jnp.concatenate([bl_mu, bl_std]).reshape(1, 2 * 64)

    out2 = pl.pallas_call(
        _dense2_body,
        grid=(_GRID,),
        in_specs=_row_specs(),
        out_specs=pl.BlockSpec((_BR, D), lambda i: (i, 0)),
        out_shape=jax.ShapeDtypeStruct((NP, D), jnp.float32),
    )(pa2, pc, h, wl2, b2, wr2)

    return out2[:NN, :64], out2[:NN, 64:]


# speedup vs baseline: 5.9814x; 1.0506x over previous
import dataclasses

import jax
import jax.numpy as jnp
from jax import lax
from jax.experimental import pallas as pl
from jax.experimental.pallas import tpu as pltpu
from jax.experimental.pallas import tpu_sc as plsc

NN = 10000
NP = 10240
D = 128
E = 320000
CHUNK = 80
NC, NS = 2, 16
NW = NC * NS
EPT = E // NW
EPT_SC = E // NS
HSLAB = EPT_SC // 2
HCH = HSLAB // CHUNK
HALF = NP // NC
TRASH = HALF
NBUF = 5
LEAD = 2
ROWS_PER_TILE = NP // NS
ZR = 16
HR = NP // D


def _mesh():
    return plsc.VectorSubcoreMesh(core_axis_name="c", subcore_axis_name="s",
                                  num_cores=NC, num_subcores=NS)


def _no_layout_params():
    cp = pltpu.CompilerParams()
    if "needs_layout_passes" in pltpu.CompilerParams.__dataclass_fields__:
        cp = dataclasses.replace(cp, needs_layout_passes=False)
    return cp


def _sc_agg_body(x_hbm, src_hbm, dst_hbm, agg_hbm, acc_sp,
                 src_half, dst_half, rows, zb_v,
                 dstb0, dstb1, dstb2, dstb3, dstb4, gsem, ssem, zsem):
    c = lax.axis_index("c")
    s = lax.axis_index("s")
    dstb = [dstb0, dstb1, dstb2, dstb3, dstb4]
    z16 = jnp.zeros((16,), jnp.float32)
    node_base = c * HALF

    def zero_zb(i, _):
        for j in range(D // 16):
            zb_v[i, pl.ds(j * 16, 16)] = z16
        return _
    lax.fori_loop(0, ZR, zero_zb, None)

    base_row = s * (HALF // NS)

    def fire_zero(i, _):
        pltpu.async_copy(zb_v, acc_sp.at[pl.ds(base_row + i * ZR, ZR)], zsem)
        return _
    lax.fori_loop(0, (HALF // NS) // ZR, fire_zero, None)

    first = True
    for half in range(EPT_SC // HSLAB):
        ebase = s * EPT_SC + half * HSLAB
        pltpu.sync_copy(src_hbm.at[pl.ds(ebase, HSLAB)], src_half)
        pltpu.sync_copy(dst_hbm.at[pl.ds(ebase, HSLAB)], dst_half)

        for b in range(LEAD):
            pltpu.async_copy(x_hbm.at[src_half.at[pl.ds(b * CHUNK, CHUNK)]],
                             rows.at[b], gsem.at[b])

        if first:
            first = False

            def drain_zero(i, _):
                pltpu.make_async_copy(
                    zb_v, acc_sp.at[pl.ds(base_row + i * ZR, ZR)],
                    zsem).wait()
                return _
            lax.fori_loop(0, (HALF // NS) // ZR, drain_zero, None)
            plsc.subcore_barrier()

        def ring_step(step, _):
            for b in range(NBUF):
                j = step * NBUF + b
                bg = (b + LEAD) % NBUF
                jg = j + LEAD

                @pl.when(jnp.logical_and(jg < HCH, jg >= NBUF))
                def _():
                    pltpu.make_async_copy(
                        rows.at[bg], acc_sp.at[dstb[bg]],
                        ssem.at[bg]).wait()

                @pl.when(jg < HCH)
                def _():
                    pltpu.async_copy(
                        x_hbm.at[src_half.at[pl.ds(jg * CHUNK, CHUNK)]],
                        rows.at[bg], gsem.at[bg])

                for v in range(CHUNK // 16):
                    idx = dst_half[pl.ds(j * CHUNK + v * 16, 16)]
                    loc = idx - node_base
                    owned = jnp.logical_and(idx >= node_base,
                                            loc < HALF)
                    dstb[b][pl.ds(v * 16, 16)] = jnp.where(
                        owned, loc, TRASH)

                pltpu.make_async_copy(
                    x_hbm.at[src_half.at[pl.ds(j * CHUNK, CHUNK)]],
                    rows.at[b], gsem.at[b]).wait()
                pltpu.async_copy(rows.at[b], acc_sp.at[dstb[b]],
                                 ssem.at[b], add=True)
            return _
        lax.fori_loop(0, HCH // NBUF, ring_step, None)

        for b in range(NBUF):
            pltpu.make_async_copy(rows.at[b], acc_sp.at[dstb[b]],
                                  ssem.at[b]).wait()

    plsc.subcore_barrier()

    pltpu.sync_copy(acc_sp.at[pl.ds(base_row, HALF // NS)],
                    agg_hbm.at[pl.ds(c * HALF + base_row, HALF // NS)])


def _sc_aggregate(x, src, dst):
    k = pl.kernel(
        _sc_agg_body,
        out_type=jax.ShapeDtypeStruct((NP, D), jnp.float32),
        mesh=_mesh(),
        scratch_types=[
            pltpu.VMEM_SHARED((HALF + 2 * ZR, D), jnp.float32),
            pltpu.VMEM((HSLAB,), jnp.int32),
            pltpu.VMEM((HSLAB,), jnp.int32),
            pltpu.VMEM((NBUF, CHUNK, D), jnp.float32),
            pltpu.VMEM((ZR, D), jnp.float32),
            pltpu.VMEM((CHUNK,), jnp.int32),
            pltpu.VMEM((CHUNK,), jnp.int32),
            pltpu.VMEM((CHUNK,), jnp.int32),
            pltpu.VMEM((CHUNK,), jnp.int32),
            pltpu.VMEM((CHUNK,), jnp.int32),
            pltpu.SemaphoreType.DMA((NBUF,)),
            pltpu.SemaphoreType.DMA((NBUF,)),
            pltpu.SemaphoreType.DMA,
        ],
    )
    return k(x, src, dst)


def _sc_count_body(dst_hbm, cnt_hbm, hist_sp, dst_v, hist_v, idx80_v, sem):
    c = lax.axis_index("c")
    s = lax.axis_index("s")
    w = c * NS + s
    z16 = jnp.zeros((16,), jnp.float32)
    ones16 = jnp.ones((16,), jnp.float32)

    def zero_hist(i, _):
        for j in range(D // 16):
            hist_v[i, pl.ds(j * 16, 16)] = z16
        return _
    lax.fori_loop(0, HR, zero_hist, None)

    def fill_idx(i, _):
        idx80_v[pl.ds(i * 16, 16)] = lax.iota(jnp.int32, 16) + i * 16
        return _
    lax.fori_loop(0, HR // 16, fill_idx, None)

    @pl.when(s < HR // 8)
    def _():
        pltpu.sync_copy(hist_v.at[pl.ds(0, 8)], hist_sp.at[pl.ds(s * 8, 8)])
    plsc.subcore_barrier()

    pltpu.sync_copy(dst_hbm.at[pl.ds(w * EPT, EPT)], dst_v)

    def edge_group(i, _):
        for j in range(8):
            idx = dst_v[pl.ds((i * 8 + j) * 16, 16)]
            row = lax.shift_right_logical(idx, 7)
            lane = lax.bitwise_and(idx, 127)
            plsc.addupdate_scatter(hist_v, [row, lane], ones16)
        return _
    lax.fori_loop(0, EPT // 128, edge_group, None)

    pltpu.sync_copy(hist_v, hist_sp.at[idx80_v], add=True)
    plsc.subcore_barrier()

    @pl.when(s == 0)
    def _():
        pltpu.sync_copy(hist_sp, cnt_hbm.at[c])


def _sc_count(dst):
    k = pl.kernel(
        _sc_count_body,
        out_type=jax.ShapeDtypeStruct((NC, HR, D), jnp.float32),
        mesh=_mesh(),
        scratch_types=[
            pltpu.VMEM_SHARED((HR, D), jnp.float32),
            pltpu.VMEM((EPT,), jnp.int32),
            pltpu.VMEM((HR, D), jnp.float32),
            pltpu.VMEM((HR,), jnp.int32),
            pltpu.SemaphoreType.DMA,
        ],
        compiler_params=_no_layout_params(),
    )
    return k(dst)


_HI = lax.Precision.HIGHEST
_BR = 1024
_GRID = NP // _BR


def _expand_cnt(pc_ref):
    pk = (pc_ref[0] + pc_ref[1]).T
    return jnp.concatenate([pk[:, g:g + 1] for g in range(_BR // D)], axis=0)


def _dense1_body(pa_ref, pc_ref, x_ref, wl_ref, bl_ref, wr_ref, h_ref):
    agg = pa_ref[...]
    cnt = _expand_cnt(pc_ref)
    mean = agg / jnp.maximum(cnt, 1.0)
    acc = jnp.dot(mean, wl_ref[...], preferred_element_type=jnp.float32,
                  precision=_HI)
    acc = acc + jnp.dot(x_ref[...], wr_ref[...],
                        preferred_element_type=jnp.float32, precision=_HI)
    h_ref[...] = jnp.maximum(acc + bl_ref[...], 0.0)


def _dense2_body(pa_ref, pc_ref, h_ref, wl_ref, b_ref, wr_ref, o_ref):
    agg = pa_ref[...]
    cnt = _expand_cnt(pc_ref)
    mean = agg / jnp.maximum(cnt, 1.0)
    acc = jnp.dot(mean, wl_ref[...], preferred_element_type=jnp.float32,
                  precision=_HI)
    acc = acc + jnp.dot(h_ref[...], wr_ref[...],
                        preferred_element_type=jnp.float32, precision=_HI)
    o_ref[...] = acc + b_ref[...]


def _row_specs():
    return [
        pl.BlockSpec((_BR, D), lambda i: (i, 0)),
        pl.BlockSpec((NC, _BR // D, D), lambda i: (0, i, 0)),
        pl.BlockSpec((_BR, D), lambda i: (i, 0)),
        pl.BlockSpec((D, D), lambda i: (0, 0)),
        pl.BlockSpec((1, D), lambda i: (0, 0)),
        pl.BlockSpec((D, D), lambda i: (0, 0)),
    ]


def kernel(features, edges, Wl1, bl1, Wr1, Wl_mu, bl_mu, Wr_mu,
           Wl_std, bl_std, Wr_std):
    edges = edges.astype(jnp.int32)
    src, dst = edges[0], edges[1]
    x = jnp.pad(features, ((0, NP - NN), (0, 0)))

    pa1 = _sc_aggregate(x, src, dst)
    pc = _sc_count(dst)

    h = pl.pallas_call(
        _dense1_body,
        grid=(_GRID,),
        in_specs=_row_specs(),
        out_specs=pl.BlockSpec((_BR, D), lambda i: (i, 0)),
        out_shape=jax.ShapeDtypeStruct((NP, D), jnp.float32),
    )(pa1, pc, x, Wl1.T, bl1.reshape(1, D), Wr1.T)

    pa2 = _sc_aggregate(h, src, dst)

    wl2 = jnp.concatenate([Wl_mu.T, Wl_std.T], axis=1)
    wr2 = jnp.concatenate([Wr_mu.T, Wr_std.T], axis=1)
    b2 = jnp.concatenate([bl_mu, bl_std]).reshape(1, 2 * 64)

    out2 = pl.pallas_call(
        _dense2_body,
        grid=(_GRID,),
        in_specs=_row_specs(),
        out_specs=pl.BlockSpec((_BR, D), lambda i: (i, 0)),
        out_shape=jax.ShapeDtypeStruct((NP, D), jnp.float32),
    )(pa2, pc, h, wl2, b2, wr2)

    return out2[:NN, :64], out2[:NN, 64:]
